# Initial kernel scaffold; baseline (speedup 1.0000x reference)
import jax
import jax.numpy as jnp
from jax.experimental import pallas as pl


def kernel(x, Wr, Wp1, Wp2, Wg, Wu, Wd):
    raise NotImplementedError("write your pallas kernel here")


# speedup vs baseline: 1.6039x; 1.6039x over previous
import functools

import jax
import jax.numpy as jnp
from jax.experimental import pallas as pl
from jax.experimental.pallas import tpu as pltpu

D_MODEL = 1024
N_SCALES = 3
N_EXP = 4
N_TOTAL = 12
TOP_K = 2
D_FFN = 2752


def _router_kernel(n_blocks, n_tok, x_ref, wr_ref, w_ref, acc_ref, aux_ref):
    i = pl.program_id(0)
    x = x_ref[...]
    logits = jnp.dot(x, wr_ref[...], preferred_element_type=jnp.float32)
    m = jnp.max(logits, axis=-1, keepdims=True)
    ex = jnp.exp(logits - m)
    probs = ex / jnp.sum(ex, axis=-1, keepdims=True)

    iota = jax.lax.broadcasted_iota(jnp.int32, probs.shape, 1)
    p1 = jnp.max(probs, axis=-1, keepdims=True)
    i1 = jnp.min(jnp.where(probs == p1, iota, N_TOTAL), axis=-1, keepdims=True)
    mask1 = iota == i1
    probs_m = jnp.where(mask1, -jnp.inf, probs)
    p2 = jnp.max(probs_m, axis=-1, keepdims=True)
    i2 = jnp.min(jnp.where(probs_m == p2, iota, N_TOTAL), axis=-1, keepdims=True)
    mask2 = iota == i2
    denom = p1 + p2 + 1e-8
    w = (jnp.where(mask1, p1, 0.0) + jnp.where(mask2, p2, 0.0)) / denom
    w_ref[...] = w

    load_part = jnp.sum(probs, axis=0, keepdims=True)
    cnt_part = jnp.sum(mask1.astype(jnp.float32) + mask2.astype(jnp.float32),
                       axis=0, keepdims=True)

    @pl.when(i == 0)
    def _():
        acc_ref[...] = jnp.zeros_like(acc_ref)

    acc_ref[0:1, :] += load_part
    acc_ref[1:2, :] += cnt_part

    @pl.when(i == n_blocks - 1)
    def _():
        load = acc_ref[0:1, :] / n_tok
        frac = acc_ref[1:2, :] / (n_tok * TOP_K)
        aux_ref[...] = (N_TOTAL * jnp.sum(frac * load)).reshape(1, 1)


def _router(x_flat, wr):
    n_tok = x_flat.shape[0]
    blk = 1024
    n_blocks = n_tok // blk
    w, _, aux = pl.pallas_call(
        functools.partial(_router_kernel, n_blocks, n_tok),
        grid=(n_blocks,),
        in_specs=[
            pl.BlockSpec((blk, D_MODEL), lambda i: (i, 0)),
            pl.BlockSpec((D_MODEL, N_TOTAL), lambda i: (0, 0)),
        ],
        out_specs=[
            pl.BlockSpec((blk, N_TOTAL), lambda i: (i, 0)),
            pl.BlockSpec((8, N_TOTAL), lambda i: (0, 0)),
            pl.BlockSpec((1, 1), lambda i: (0, 0)),
        ],
        out_shape=[
            jax.ShapeDtypeStruct((n_tok, N_TOTAL), jnp.float32),
            jax.ShapeDtypeStruct((8, N_TOTAL), jnp.float32),
            jax.ShapeDtypeStruct((1, 1), jnp.float32),
        ],
    )(x_flat, wr)
    return w, aux


def _pool_kernel(n_k, x_ref, wp_ref, o_ref, acc_ref):
    k = pl.program_id(0)

    @pl.when(k == 0)
    def _():
        acc_ref[...] = jnp.zeros_like(acc_ref)

    xb = x_ref[...].astype(jnp.bfloat16)
    wb = wp_ref[...].astype(jnp.bfloat16)
    acc_ref[...] += jnp.dot(xb, wb, preferred_element_type=jnp.float32)

    @pl.when(k == n_k - 1)
    def _():
        o_ref[...] = acc_ref[...].astype(jnp.bfloat16)


def _pool(x_grouped, wp, kb):
    m, kdim = x_grouped.shape
    n_k = kdim // kb
    return pl.pallas_call(
        functools.partial(_pool_kernel, n_k),
        grid=(n_k,),
        in_specs=[
            pl.BlockSpec((m, kb), lambda k: (0, k)),
            pl.BlockSpec((kb, D_MODEL), lambda k: (k, 0)),
        ],
        out_specs=pl.BlockSpec((m, D_MODEL), lambda k: (0, 0)),
        out_shape=jax.ShapeDtypeStruct((m, D_MODEL), jnp.bfloat16),
        scratch_shapes=[pltpu.VMEM((m, D_MODEL), jnp.float32)],
    )(x_grouped, wp)


def _ffn_kernel(n_blocks, nb, r_ref, wg_ref, wu_ref, wd_ref, o_ref):
    n = pl.program_id(1)
    lim = D_FFN - n * nb
    col = jax.lax.broadcasted_iota(jnp.int32, (D_MODEL, nb), 1)
    row = jax.lax.broadcasted_iota(jnp.int32, (nb, D_MODEL), 0)
    wg = jnp.where(col < lim, wg_ref[0, 0], 0.0).astype(jnp.bfloat16)
    wu = jnp.where(col < lim, wu_ref[0, 0], 0.0).astype(jnp.bfloat16)
    wd = jnp.where(row < lim, wd_ref[0, 0], 0.0).astype(jnp.bfloat16)

    r = r_ref[...]
    a = jnp.dot(r, wg, preferred_element_type=jnp.float32)
    b = jnp.dot(r, wu, preferred_element_type=jnp.float32)
    h = (a * jax.nn.sigmoid(a) * b).astype(jnp.bfloat16)
    part = jnp.dot(h, wd, preferred_element_type=jnp.float32)

    @pl.when(n == 0)
    def _():
        o_ref[...] = jnp.zeros_like(o_ref)

    o_ref[0] += part


def _ffn_experts(r, wg, wu, wd, scale_idx, nb):
    m = r.shape[0]
    n_blocks = (D_FFN + nb - 1) // nb
    return pl.pallas_call(
        functools.partial(_ffn_kernel, n_blocks, nb),
        grid=(N_EXP, n_blocks),
        in_specs=[
            pl.BlockSpec((m, D_MODEL), lambda e, n: (0, 0)),
            pl.BlockSpec((1, 1, D_MODEL, nb), lambda e, n: (scale_idx, e, 0, n)),
            pl.BlockSpec((1, 1, D_MODEL, nb), lambda e, n: (scale_idx, e, 0, n)),
            pl.BlockSpec((1, 1, nb, D_MODEL), lambda e, n: (scale_idx, e, n, 0)),
        ],
        out_specs=pl.BlockSpec((1, m, D_MODEL), lambda e, n: (e, 0, 0)),
        out_shape=jax.ShapeDtypeStruct((N_EXP, m, D_MODEL), jnp.float32),
    )(r, wg, wu, wd)


def _ffn0_kernel(n_blocks, nb, x_ref, w_ref, wg_ref, wu_ref, wd_ref, o_ref):
    e = pl.program_id(0)
    n = pl.program_id(1)
    lim = D_FFN - n * nb
    col = jax.lax.broadcasted_iota(jnp.int32, (D_MODEL, nb), 1)
    row = jax.lax.broadcasted_iota(jnp.int32, (nb, D_MODEL), 0)
    wg = jnp.where(col < lim, wg_ref[0, 0], 0.0).astype(jnp.bfloat16)
    wu = jnp.where(col < lim, wu_ref[0, 0], 0.0).astype(jnp.bfloat16)
    wd = jnp.where(row < lim, wd_ref[0, 0], 0.0).astype(jnp.bfloat16)

    x = x_ref[...]
    a = jnp.dot(x, wg, preferred_element_type=jnp.float32)
    b = jnp.dot(x, wu, preferred_element_type=jnp.float32)
    h = (a * jax.nn.sigmoid(a) * b).astype(jnp.bfloat16)
    part = jnp.dot(h, wd, preferred_element_type=jnp.float32)
    wmat = w_ref[...]
    lane = jax.lax.broadcasted_iota(jnp.int32, wmat.shape, 1)
    wcol = jnp.sum(jnp.where(lane == e, wmat, 0.0), axis=1, keepdims=True)

    @pl.when(jnp.logical_and(e == 0, n == 0))
    def _():
        o_ref[...] = jnp.zeros_like(o_ref)

    o_ref[...] += wcol * part


def _ffn0(xb, w, wg, wu, wd, nb):
    m = xb.shape[0]
    n_blocks = (D_FFN + nb - 1) // nb
    return pl.pallas_call(
        functools.partial(_ffn0_kernel, n_blocks, nb),
        grid=(N_EXP, n_blocks),
        in_specs=[
            pl.BlockSpec((m, D_MODEL), lambda e, n: (0, 0)),
            pl.BlockSpec((m, N_TOTAL), lambda e, n: (0, 0)),
            pl.BlockSpec((1, 1, D_MODEL, nb), lambda e, n: (0, e, 0, n)),
            pl.BlockSpec((1, 1, D_MODEL, nb), lambda e, n: (0, e, 0, n)),
            pl.BlockSpec((1, 1, nb, D_MODEL), lambda e, n: (0, e, n, 0)),
        ],
        out_specs=pl.BlockSpec((m, D_MODEL), lambda e, n: (0, 0)),
        out_shape=jax.ShapeDtypeStruct((m, D_MODEL), jnp.float32),
    )(xb, w, wg, wu, wd)


def _combine_kernel(blk, o0_ref, e1_ref, e2_ref, w_ref, out_ref):
    acc = o0_ref[...]
    w = w_ref[...]
    g1 = blk // 4
    g2 = blk // 16
    for e in range(N_EXP):
        v1 = e1_ref[e]
        v1r = jnp.broadcast_to(v1[:, None, :], (g1, 4, D_MODEL)).reshape(blk, D_MODEL)
        acc += w[:, N_EXP + e:N_EXP + e + 1] * v1r
        v2 = e2_ref[e]
        v2r = jnp.broadcast_to(v2[:, None, :], (g2, 16, D_MODEL)).reshape(blk, D_MODEL)
        acc += w[:, 2 * N_EXP + e:2 * N_EXP + e + 1] * v2r
    out_ref[...] = acc


def _combine(o0, e1, e2, w):
    n_tok = o0.shape[0]
    blk = 1024
    n_blocks = n_tok // blk
    return pl.pallas_call(
        functools.partial(_combine_kernel, blk),
        grid=(n_blocks,),
        in_specs=[
            pl.BlockSpec((blk, D_MODEL), lambda i: (i, 0)),
            pl.BlockSpec((N_EXP, blk // 4, D_MODEL), lambda i: (0, i, 0)),
            pl.BlockSpec((N_EXP, blk // 16, D_MODEL), lambda i: (0, i, 0)),
            pl.BlockSpec((blk, N_TOTAL), lambda i: (i, 0)),
        ],
        out_specs=pl.BlockSpec((blk, D_MODEL), lambda i: (i, 0)),
        out_shape=jax.ShapeDtypeStruct((n_tok, D_MODEL), jnp.float32),
    )(o0, e1, e2, w)


def kernel(x, Wr, Wp1, Wp2, Wg, Wu, Wd):
    B, T, D = x.shape
    n_tok = B * T
    x_flat = x.reshape(n_tok, D)
    x1 = x.reshape(n_tok // 4, 4 * D)
    x2 = x.reshape(n_tok // 16, 16 * D)

    w, aux = _router(x_flat, Wr)
    r1 = _pool(x1, Wp1, 1024)
    r2 = _pool(x2, Wp2, 2048)
    e1 = _ffn_experts(r1, Wg, Wu, Wd, 1, 512)
    e2 = _ffn_experts(r2, Wg, Wu, Wd, 2, 512)
    o0 = _ffn0(x_flat.astype(jnp.bfloat16), w, Wg, Wu, Wd, 256)
    out = _combine(o0, e1, e2, w)
    return out.reshape(B, T, D), aux.reshape(())
